# Initial kernel scaffold; baseline (speedup 1.0000x reference)
import jax
import jax.numpy as jnp
from jax.experimental import pallas as pl


def kernel(pedestal_table, spatial_embeddings, pedestals):
    raise NotImplementedError("write your pallas kernel here")


# speedup vs baseline: 2.1149x; 2.1149x over previous
import dataclasses
import functools

import jax
import jax.numpy as jnp
from jax import lax
from jax.experimental import pallas as pl
from jax.experimental.pallas import tpu as pltpu
from jax.experimental.pallas import tpu_sc as plsc

N_CH = 1048576
NUM_PED = 16
PED_F = 4
SPA_F = 2
OUT_F = PED_F + SPA_F

NC, NS, L = 2, 16, 16
NW = NC * NS
CH_PER_W = N_CH // NW
W = 8192
N_BLK = CH_PER_W // W


def _body(table_hbm, spatial_hbm, ped_hbm, out_hbm,
          table_v, idx_v, s_v, out_v):
    wid = lax.axis_index("s") * NC + lax.axis_index("c")
    w_base = wid * CH_PER_W

    pltpu.sync_copy(table_hbm, table_v)

    lanes = lax.iota(jnp.int32, L)

    @pl.loop(0, N_BLK)
    def _(b):
        base = w_base + b * W
        pltpu.sync_copy(ped_hbm.at[pl.ds(base, W)], idx_v)
        pltpu.sync_copy(spatial_hbm.at[pl.ds(base * SPA_F, W * SPA_F)], s_v)

        @pl.loop(0, W // L)
        def _(j):
            c_loc = lanes + j * L
            o_base = c_loc * OUT_F
            p4 = idx_v[pl.ds(j * L, L)] * PED_F
            s_base = c_loc * SPA_F
            for f in range(PED_F):
                vals = plsc.load_gather(table_v, [p4 + f])
                plsc.store_scatter(out_v, [o_base + f], vals)
            for f in range(SPA_F):
                vals = plsc.load_gather(s_v, [s_base + f])
                plsc.store_scatter(out_v, [o_base + (PED_F + f)], vals)

        pltpu.sync_copy(out_v, out_hbm.at[pl.ds(base * OUT_F, W * OUT_F)])


def kernel(pedestal_table, spatial_embeddings, pedestals):
    mesh = plsc.VectorSubcoreMesh(core_axis_name="c", subcore_axis_name="s")
    cp = pltpu.CompilerParams()
    if "needs_layout_passes" in pltpu.CompilerParams.__dataclass_fields__:
        cp = dataclasses.replace(cp, needs_layout_passes=False)
    k = functools.partial(
        pl.kernel,
        out_type=jax.ShapeDtypeStruct((N_CH * OUT_F,), jnp.float32),
        mesh=mesh,
        scratch_types=[
            pltpu.VMEM((NUM_PED * PED_F,), jnp.float32),
            pltpu.VMEM((W,), jnp.int32),
            pltpu.VMEM((W * SPA_F,), jnp.float32),
            pltpu.VMEM((W * OUT_F,), jnp.float32),
        ],
        compiler_params=cp,
    )(_body)
    out_flat = k(pedestal_table.reshape(NUM_PED * PED_F),
                 spatial_embeddings.reshape(N_CH * SPA_F),
                 pedestals)
    return out_flat.reshape(N_CH, OUT_F)


# speedup vs baseline: 41.8602x; 19.7932x over previous
import dataclasses
import functools

import jax
import jax.numpy as jnp
from jax import lax
from jax.experimental import pallas as pl
from jax.experimental.pallas import tpu as pltpu
from jax.experimental.pallas import tpu_sc as plsc

N_CH = 1048576
NUM_PED = 16
PED_F = 4
SPA_F = 2
OUT_F = PED_F + SPA_F
OUT_R = 8

NC, NS, L = 2, 16, 16
NW = NC * NS
N_CHUNK = N_CH // 128
CHUNK_PER_W = N_CHUNK // NW
WG = 64
N_BLK = CHUNK_PER_W // WG
REP_STRIDE = 65


def _body(table_hbm, spatial_hbm, ped_hbm, out_hbm, table_v, rep_v, idx_v, out_v):
    wid = lax.axis_index("s") * NC + lax.axis_index("c")
    w_base = wid * CHUNK_PER_W

    lanes = lax.iota(jnp.int32, L)

    pltpu.sync_copy(table_hbm, table_v)

    @pl.loop(0, L)
    def _(i):
        @pl.loop(0, NUM_PED * PED_F // L)
        def _(k):
            v = table_v[pl.ds(k * L, L)]
            rep_v[pl.ds(i * REP_STRIDE + k * L, L)] = v

    lane_off = lanes * REP_STRIDE

    @pl.loop(0, N_BLK)
    def _(b):
        g0 = w_base + b * WG
        pltpu.sync_copy(ped_hbm.at[pl.ds(g0 * 128, WG * 128)], idx_v)
        pltpu.sync_copy(spatial_hbm.at[pl.ds(g0, WG)],
                        out_v.at[:, PED_F:PED_F + SPA_F, :])

        @pl.loop(0, WG)
        def _(c):
            for s in range(128 // L):
                p = idx_v[pl.ds(c * 128 + s * L, L)]
                a = p * PED_F + lane_off
                for f in range(PED_F):
                    vals = plsc.load_gather(rep_v, [a + f])
                    out_v.at[c, f][pl.ds(s * L, L)] = vals

        pltpu.sync_copy(out_v, out_hbm.at[pl.ds(g0, WG)])


def kernel(pedestal_table, spatial_embeddings, pedestals):
    mesh = plsc.VectorSubcoreMesh(core_axis_name="c", subcore_axis_name="s")
    cp = pltpu.CompilerParams()
    if "needs_layout_passes" in pltpu.CompilerParams.__dataclass_fields__:
        cp = dataclasses.replace(cp, needs_layout_passes=False)
    k = functools.partial(
        pl.kernel,
        out_type=jax.ShapeDtypeStruct((N_CHUNK, OUT_R, 128), jnp.float32),
        mesh=mesh,
        scratch_types=[
            pltpu.VMEM((NUM_PED * PED_F,), jnp.float32),
            pltpu.VMEM((REP_STRIDE * L,), jnp.float32),
            pltpu.VMEM((WG * 128,), jnp.int32),
            pltpu.VMEM((WG, OUT_R, 128), jnp.float32),
        ],
        compiler_params=cp,
    )(_body)
    spatial3 = spatial_embeddings.reshape(N_CHUNK, 128, SPA_F).transpose(0, 2, 1)
    out3 = k(pedestal_table.reshape(NUM_PED * PED_F), spatial3, pedestals)
    return out3.transpose(0, 2, 1)[:, :, :OUT_F].reshape(N_CH, OUT_F)


# speedup vs baseline: 52.5835x; 1.2562x over previous
import dataclasses
import functools

import jax
import jax.numpy as jnp
from jax import lax
from jax.experimental import pallas as pl
from jax.experimental.pallas import tpu as pltpu
from jax.experimental.pallas import tpu_sc as plsc

N_CH = 1048576
NUM_PED = 16
PED_F = 4
SPA_F = 2
OUT_F = PED_F + SPA_F
OUT_R = 8

NC, NS, L = 2, 16, 16
NW = NC * NS
N_CHUNK = N_CH // 128
CHUNK_PER_W = N_CHUNK // NW
WG = 64
N_BLK = CHUNK_PER_W // WG
REP_STRIDE = 65


def _body(table_hbm, spatial_hbm, ped_hbm, out_hbm, table_v, rep_v, idx_v, out_v):
    wid = lax.axis_index("s") * NC + lax.axis_index("c")
    w_base = wid * CHUNK_PER_W

    lanes = lax.iota(jnp.int32, L)

    pltpu.sync_copy(table_hbm, table_v)

    @pl.loop(0, L)
    def _(i):
        @pl.loop(0, NUM_PED * PED_F // L)
        def _(k):
            v = table_v[pl.ds(k * L, L)]
            rep_v[pl.ds(i * REP_STRIDE + k * L, L)] = v

    lane_off = lanes * REP_STRIDE

    @pl.loop(0, N_BLK)
    def _(b):
        g0 = w_base + b * WG
        pltpu.sync_copy(ped_hbm.at[pl.ds(g0 * 128, WG * 128)], idx_v)
        pltpu.sync_copy(spatial_hbm.at[pl.ds(g0, WG)],
                        out_v.at[:, PED_F:PED_F + SPA_F, :])

        @pl.loop(0, WG)
        def _(c):
            for s in range(128 // L):
                p = idx_v[pl.ds(c * 128 + s * L, L)]
                a = p * PED_F + lane_off
                for f in range(PED_F):
                    vals = plsc.load_gather(rep_v, [a + f])
                    out_v.at[c, f][pl.ds(s * L, L)] = vals

        pltpu.sync_copy(out_v, out_hbm.at[pl.ds(g0, WG)])


def kernel(pedestal_table, spatial_embeddings, pedestals):
    mesh = plsc.VectorSubcoreMesh(core_axis_name="c", subcore_axis_name="s")
    cp = pltpu.CompilerParams()
    if "needs_layout_passes" in pltpu.CompilerParams.__dataclass_fields__:
        cp = dataclasses.replace(cp, needs_layout_passes=False)
    k = functools.partial(
        pl.kernel,
        out_type=jax.ShapeDtypeStruct((N_CHUNK, OUT_R, 128), jnp.float32),
        mesh=mesh,
        scratch_types=[
            pltpu.VMEM((NUM_PED * PED_F,), jnp.float32),
            pltpu.VMEM((REP_STRIDE * L,), jnp.float32),
            pltpu.VMEM((WG * 128,), jnp.int32),
            pltpu.VMEM((WG, OUT_R, 128), jnp.float32),
        ],
        compiler_params=cp,
    )(_body)
    spatial3 = spatial_embeddings.reshape(N_CHUNK, 128, SPA_F).transpose(0, 2, 1)
    out3 = k(pedestal_table.reshape(NUM_PED * PED_F), spatial3, pedestals)
    return out3.transpose(0, 2, 1).reshape(N_CH, OUT_R)[:, :OUT_F]


# speedup vs baseline: 54.8512x; 1.0431x over previous
import dataclasses
import functools

import jax
import jax.numpy as jnp
from jax import lax
from jax.experimental import pallas as pl
from jax.experimental.pallas import tpu as pltpu
from jax.experimental.pallas import tpu_sc as plsc

N_CH = 1048576
NUM_PED = 16
PED_F = 4
SPA_F = 2
OUT_F = PED_F + SPA_F
OUT_R = 8

NC, NS, L = 2, 16, 16
NW = NC * NS
N_CHUNK = N_CH // 128
CHUNK_PER_W = N_CHUNK // NW
WG = 32
N_BLK = CHUNK_PER_W // WG


def _body(table_hbm, spatial_hbm, ped_hbm, out_hbm,
          table_v, rep_v, idx_v, out_v0, out_v1, sem0, sem1):
    wid = lax.axis_index("s") * NC + lax.axis_index("c")
    w_base = wid * CHUNK_PER_W

    lanes = lax.iota(jnp.int32, L)

    pltpu.sync_copy(table_hbm, table_v)
    for k in range(NUM_PED * PED_F):
        v = plsc.load_gather(table_v, [jnp.full((L,), k, jnp.int32)])
        rep_v[pl.ds(k * L, L)] = v

    out_bufs = (out_v0, out_v1)
    sems = (sem0, sem1)
    out_dma = [None, None]

    for blk in range(N_BLK):
        par = blk % 2
        out_v = out_bufs[par]
        g0 = w_base + blk * WG
        pltpu.sync_copy(ped_hbm.at[pl.ds(g0 * 128, WG * 128)], idx_v)
        if out_dma[par] is not None:
            out_dma[par].wait()
        pltpu.sync_copy(spatial_hbm.at[pl.ds(g0, WG)],
                        out_v.at[:, PED_F:PED_F + SPA_F, :])

        @pl.loop(0, WG)
        def _(c):
            for s in range(128 // L):
                p = idx_v[pl.ds(c * 128 + s * L, L)]
                a = p * (L * PED_F) + lanes
                for f in range(PED_F):
                    vals = plsc.load_gather(rep_v, [a + f * L])
                    out_v.at[c, f][pl.ds(s * L, L)] = vals

        out_dma[par] = pltpu.async_copy(out_v, out_hbm.at[pl.ds(g0, WG)],
                                        sems[par])

    for d in out_dma:
        d.wait()


def kernel(pedestal_table, spatial_embeddings, pedestals):
    mesh = plsc.VectorSubcoreMesh(core_axis_name="c", subcore_axis_name="s")
    cp = pltpu.CompilerParams()
    if "needs_layout_passes" in pltpu.CompilerParams.__dataclass_fields__:
        cp = dataclasses.replace(cp, needs_layout_passes=False)
    k = functools.partial(
        pl.kernel,
        out_type=jax.ShapeDtypeStruct((N_CHUNK, OUT_R, 128), jnp.float32),
        mesh=mesh,
        scratch_types=[
            pltpu.VMEM((NUM_PED * PED_F,), jnp.float32),
            pltpu.VMEM((NUM_PED * PED_F * L,), jnp.float32),
            pltpu.VMEM((WG * 128,), jnp.int32),
            pltpu.VMEM((WG, OUT_R, 128), jnp.float32),
            pltpu.VMEM((WG, OUT_R, 128), jnp.float32),
            pltpu.SemaphoreType.DMA,
            pltpu.SemaphoreType.DMA,
        ],
        compiler_params=cp,
    )(_body)
    spatial3 = spatial_embeddings.reshape(N_CHUNK, 128, SPA_F).transpose(0, 2, 1)
    out3 = k(pedestal_table.reshape(NUM_PED * PED_F), spatial3, pedestals)
    return out3.transpose(0, 2, 1).reshape(N_CH, OUT_R)[:, :OUT_F]
